# baseline (speedup 1.0000x reference)
import functools

import jax
import jax.numpy as jnp
from jax import lax
from jax.experimental import pallas as pl
from jax.experimental.pallas import tpu as pltpu
from jax.experimental.pallas import tpu_sc as plsc


def _tc_linearize(wt, V, D):
    R = 1024
    rows = V * D // 128
    nb = -(-rows // R)

    def body(in_ref, out_ref):
        x = in_ref[...]
        y = jnp.transpose(x, (1, 0))
        z = y.reshape(R, 128 // D, D)
        out_ref[...] = jnp.concatenate(
            [z[:, q, :] for q in range(128 // D)], axis=1)

    return pl.pallas_call(
        body,
        grid=(nb,),
        in_specs=[pl.BlockSpec((D, (128 // D) * R), lambda i: (0, i))],
        out_specs=pl.BlockSpec((R, 128), lambda i: (i, 0)),
        out_shape=jax.ShapeDtypeStruct((rows, 128), jnp.float32),
    )(wt)


def _gather_shuffle_kernel(B, F, D, NC, NW):
    mesh = plsc.VectorSubcoreMesh(core_axis_name="c", subcore_axis_name="s")
    L = 16
    BB = 512
    n_units = F * (B // BB)
    upw = n_units // NW
    NBUF = 2

    @functools.partial(
        pl.kernel,
        mesh=mesh,
        out_type=jax.ShapeDtypeStruct((F, D, B), jnp.float32),
        scratch_types=[
            [pltpu.VMEM((BB,), jnp.int32) for _ in range(NBUF)],
            [pltpu.VMEM((BB, D), jnp.float32) for _ in range(NBUF)],
            [pltpu.VMEM((D, BB), jnp.float32) for _ in range(NBUF)],
            [pltpu.SemaphoreType.DMA for _ in range(NBUF)],
            [pltpu.SemaphoreType.DMA for _ in range(NBUF)],
            [pltpu.SemaphoreType.DMA for _ in range(NBUF)],
        ],
        compiler_params=pltpu.CompilerParams(
            use_tc_tiling_on_sc=False, needs_layout_passes=False),
    )
    def k(table_hbm, idx_hbm, out_hbm, idx_v, rows_v, tr_v, sem_i, sem_g, sem_o):
        wid = lax.axis_index("s") * NC + lax.axis_index("c")
        u0 = wid * upw
        viota = lax.iota(jnp.int32, L)
        NBB = B // BB
        TPB = BB // L
        ng = upw // NBUF

        def unit_off(u):
            return u // NBB, (u % NBB) * BB

        def start_idx(u, b):
            f, b0 = unit_off(u)
            pltpu.async_copy(idx_hbm.at[pl.ds(f * B + b0, BB)], idx_v[b],
                             sem_i[b])

        def wait_idx(b):
            pltpu.make_async_copy(
                idx_hbm.at[pl.ds(0, BB)], idx_v[b], sem_i[b]).wait()

        def wait_out(b):
            pltpu.make_async_copy(
                tr_v[b], out_hbm.at[0, :, pl.ds(0, BB)], sem_o[b]).wait()

        rots = [(j + viota) % L for j in range(L)]

        def shuffle(b):
            rv, tv = rows_v[b], tr_v[b]
            for dg in range(D // L):
                for j in range(L):
                    colg = dg * L + rots[j]

                    def shuf(t, c):
                        rowg = t * L + viota
                        vals = plsc.load_gather(rv, [rowg, colg])
                        plsc.store_scatter(tv, [colg, rowg], vals)
                        return c

                    lax.fori_loop(0, TPB, shuf, 0, unroll=8)

        def sgather(b):
            pltpu.async_copy(table_hbm.at[idx_v[b]], rows_v[b], sem_g[b])

        def wait_gather(b):
            pltpu.make_async_copy(
                table_hbm.at[idx_v[b]], rows_v[b], sem_g[b]).wait()

        def emit(b, u):
            f, b0 = unit_off(u)
            shuffle(b)
            pltpu.async_copy(tr_v[b], out_hbm.at[f, :, pl.ds(b0, BB)],
                             sem_o[b])

        end = u0 + upw
        start_idx(u0, 0)
        wait_idx(0)
        sgather(0)
        start_idx(u0 + 1, 1)

        def body(g, c):
            u = u0 + NBUF * g
            wait_gather(0)

            @pl.when(u + 2 < end)
            def _():
                start_idx(u + 2, 0)

            wait_idx(1)

            @pl.when(g > 0)
            def _():
                wait_out(1)

            sgather(1)
            emit(0, u)
            wait_gather(1)

            @pl.when(u + 3 < end)
            def _():
                start_idx(u + 3, 1)

            @pl.when(g + 1 < ng)
            def _():
                wait_idx(0)
                wait_out(0)
                sgather(0)

            emit(1, u + 1)
            return c

        lax.fori_loop(0, ng, body, 0)
        wait_out(0)
        wait_out(1)

    return k


def _format_kernel(B, F, D, NC, NW):
    mesh = plsc.VectorSubcoreMesh(core_axis_name="c", subcore_axis_name="s")
    CW = 4096
    n_units = F * (D // 8) * (B // CW)
    upw = n_units // NW

    @functools.partial(
        pl.kernel,
        mesh=mesh,
        out_type=jax.ShapeDtypeStruct((F, D, B), jnp.float32),
        scratch_types=[
            [pltpu.VMEM((8, CW), jnp.float32) for _ in range(2)],
            [pltpu.SemaphoreType.DMA for _ in range(2)],
            [pltpu.SemaphoreType.DMA for _ in range(2)],
        ],
    )
    def k(g_hbm, out_hbm, buf, sem_i, sem_o):
        wid = lax.axis_index("s") * NC + lax.axis_index("c")
        u0 = wid * upw

        out_cp = [None, None]
        for j in range(upw):
            bu = j % 2
            u = u0 + j
            f = u // ((D // 8) * (B // CW))
            r = u % ((D // 8) * (B // CW))
            tr = r // (B // CW)
            c0 = (r % (B // CW)) * CW
            if out_cp[bu] is not None:
                out_cp[bu].wait()
            cps = [pltpu.async_copy(
                g_hbm.at[pl.ds(((f * D) + 8 * tr + s) * B + c0, CW)],
                buf[bu].at[s], sem_i[bu]) for s in range(8)]
            for c in cps:
                c.wait()
            out_cp[bu] = pltpu.async_copy(
                buf[bu], out_hbm.at[f, pl.ds(8 * tr, 8), pl.ds(c0, CW)],
                sem_o[bu])
        for bu in range(2):
            if out_cp[bu] is not None:
                out_cp[bu].wait()

    return k


def kernel(weight, indices, padding_idx, scale_grad_by_freq, sparse, out):
    B, F = indices.shape
    V, D = weight.shape
    N = B * F

    info = plsc.get_sparse_core_info()
    NC, NS = info.num_cores, info.num_subcores
    NW = NC * NS

    idx_flat = jnp.swapaxes(indices, 0, 1).reshape(N)
    w128 = _tc_linearize(jnp.swapaxes(weight, 0, 1), V, D)
    g3 = _gather_shuffle_kernel(B, F, D, NC, NW)(w128.reshape(V, D), idx_flat)
    res = _format_kernel(B, F, D, NC, NW)(g3.reshape(N * D))
    return jnp.transpose(res, (2, 0, 1))


# speedup vs baseline: 1.0122x; 1.0122x over previous
import functools

import jax
import jax.numpy as jnp
from jax import lax
from jax.experimental import pallas as pl
from jax.experimental.pallas import tpu as pltpu
from jax.experimental.pallas import tpu_sc as plsc


def _tc_linearize(wt, V, D):
    R = 1024
    rows = V * D // 128
    nb = -(-rows // R)

    def body(in_ref, out_ref):
        x = in_ref[...]
        y = jnp.transpose(x, (1, 0))
        z = y.reshape(R, 128 // D, D)
        out_ref[...] = jnp.concatenate(
            [z[:, q, :] for q in range(128 // D)], axis=1)

    return pl.pallas_call(
        body,
        grid=(nb,),
        in_specs=[pl.BlockSpec((D, (128 // D) * R), lambda i: (0, i))],
        out_specs=pl.BlockSpec((R, 128), lambda i: (i, 0)),
        out_shape=jax.ShapeDtypeStruct((rows, 128), jnp.float32),
    )(wt)


def _gather_shuffle_kernel(B, F, D, NC, NW):
    mesh = plsc.VectorSubcoreMesh(core_axis_name="c", subcore_axis_name="s")
    L = 16
    BB = 512
    n_units = F * (B // BB)
    upw = n_units // NW
    NBUF = 2

    @functools.partial(
        pl.kernel,
        mesh=mesh,
        out_type=jax.ShapeDtypeStruct((F, D, B), jnp.float32),
        scratch_types=[
            [pltpu.VMEM((BB,), jnp.int32) for _ in range(NBUF)],
            [pltpu.VMEM((BB, D), jnp.float32) for _ in range(NBUF)],
            [pltpu.VMEM((D, BB), jnp.float32) for _ in range(NBUF)],
            [pltpu.SemaphoreType.DMA for _ in range(NBUF)],
            [pltpu.SemaphoreType.DMA for _ in range(NBUF)],
            [pltpu.SemaphoreType.DMA for _ in range(NBUF)],
        ],
        compiler_params=pltpu.CompilerParams(
            use_tc_tiling_on_sc=False, needs_layout_passes=False),
    )
    def k(table_hbm, idx_hbm, out_hbm, idx_v, rows_v, tr_v, sem_i, sem_g, sem_o):
        wid = lax.axis_index("s") * NC + lax.axis_index("c")
        u0 = wid * upw
        viota = lax.iota(jnp.int32, L)
        NBB = B // BB
        TPB = BB // L
        ng = upw // NBUF

        def unit_off(u):
            return u // NBB, (u % NBB) * BB

        def start_idx(u, b):
            f, b0 = unit_off(u)
            pltpu.async_copy(idx_hbm.at[pl.ds(f * B + b0, BB)], idx_v[b],
                             sem_i[b])

        def wait_idx(b):
            pltpu.make_async_copy(
                idx_hbm.at[pl.ds(0, BB)], idx_v[b], sem_i[b]).wait()

        def wait_out(b):
            pltpu.make_async_copy(
                tr_v[b], out_hbm.at[0, :, pl.ds(0, BB)], sem_o[b]).wait()

        rots = [(j + viota) % L for j in range(L)]

        def shuffle(b):
            rv, tv = rows_v[b], tr_v[b]
            for dg in range(D // L):
                for j in range(L):
                    colg = dg * L + rots[j]

                    def shuf(t, c):
                        rowg = t * L + viota
                        vals = plsc.load_gather(rv, [rowg, colg])
                        plsc.store_scatter(tv, [colg, rowg], vals)
                        return c

                    lax.fori_loop(0, TPB, shuf, 0, unroll=4)

        def sgather(b):
            pltpu.async_copy(table_hbm.at[idx_v[b]], rows_v[b], sem_g[b])

        def wait_gather(b):
            pltpu.make_async_copy(
                table_hbm.at[idx_v[b]], rows_v[b], sem_g[b]).wait()

        def emit(b, u):
            f, b0 = unit_off(u)
            shuffle(b)
            pltpu.async_copy(tr_v[b], out_hbm.at[f, :, pl.ds(b0, BB)],
                             sem_o[b])

        end = u0 + upw
        start_idx(u0, 0)
        wait_idx(0)
        sgather(0)
        start_idx(u0 + 1, 1)

        def body(g, c):
            u = u0 + NBUF * g
            wait_gather(0)

            @pl.when(u + 2 < end)
            def _():
                start_idx(u + 2, 0)

            wait_idx(1)

            @pl.when(g > 0)
            def _():
                wait_out(1)

            sgather(1)
            emit(0, u)
            wait_gather(1)

            @pl.when(u + 3 < end)
            def _():
                start_idx(u + 3, 1)

            @pl.when(g + 1 < ng)
            def _():
                wait_idx(0)
                wait_out(0)
                sgather(0)

            emit(1, u + 1)
            return c

        lax.fori_loop(0, ng, body, 0)
        wait_out(0)
        wait_out(1)

    return k


def _format_kernel(B, F, D, NC, NW):
    mesh = plsc.VectorSubcoreMesh(core_axis_name="c", subcore_axis_name="s")
    CW = 4096
    n_units = F * (D // 8) * (B // CW)
    upw = n_units // NW

    @functools.partial(
        pl.kernel,
        mesh=mesh,
        out_type=jax.ShapeDtypeStruct((F, D, B), jnp.float32),
        scratch_types=[
            [pltpu.VMEM((8, CW), jnp.float32) for _ in range(2)],
            [pltpu.SemaphoreType.DMA for _ in range(2)],
            [pltpu.SemaphoreType.DMA for _ in range(2)],
        ],
    )
    def k(g_hbm, out_hbm, buf, sem_i, sem_o):
        wid = lax.axis_index("s") * NC + lax.axis_index("c")
        u0 = wid * upw

        out_cp = [None, None]
        for j in range(upw):
            bu = j % 2
            u = u0 + j
            f = u // ((D // 8) * (B // CW))
            r = u % ((D // 8) * (B // CW))
            tr = r // (B // CW)
            c0 = (r % (B // CW)) * CW
            if out_cp[bu] is not None:
                out_cp[bu].wait()
            cps = [pltpu.async_copy(
                g_hbm.at[pl.ds(((f * D) + 8 * tr + s) * B + c0, CW)],
                buf[bu].at[s], sem_i[bu]) for s in range(8)]
            for c in cps:
                c.wait()
            out_cp[bu] = pltpu.async_copy(
                buf[bu], out_hbm.at[f, pl.ds(8 * tr, 8), pl.ds(c0, CW)],
                sem_o[bu])
        for bu in range(2):
            if out_cp[bu] is not None:
                out_cp[bu].wait()

    return k


def kernel(weight, indices, padding_idx, scale_grad_by_freq, sparse, out):
    B, F = indices.shape
    V, D = weight.shape
    N = B * F

    info = plsc.get_sparse_core_info()
    NC, NS = info.num_cores, info.num_subcores
    NW = NC * NS

    idx_flat = jnp.swapaxes(indices, 0, 1).reshape(N)
    w128 = _tc_linearize(jnp.swapaxes(weight, 0, 1), V, D)
    g3 = _gather_shuffle_kernel(B, F, D, NC, NW)(w128.reshape(V, D), idx_flat)
    res = _format_kernel(B, F, D, NC, NW)(g3.reshape(N * D))
    return jnp.transpose(res, (2, 0, 1))
